# baseline (speedup 1.0000x reference)
import functools

import jax
import jax.numpy as jnp
from jax import lax
from jax.experimental import pallas as pl
from jax.experimental.pallas import tpu as pltpu
from jax.experimental.pallas import tpu_sc as plsc

E = 320000
NA = 10000
MAX_NB = 32
BOND_FDIM = 144
ATOM_FDIM = 128
H = 128
DEPTH = 5
N_MOLS = 200
MOL_SIZE = 50

NC = 2
NS = 16
NW = NC * NS

A_BLK = 4
IDX_BLK = A_BLK * MAX_NB
GS_NBLK = NA // A_BLK
GS_K = 80
GS_PAD = NW * GS_K * IDX_BLK
GS_RING = 4


def _sc_gather_sum_body(msg_hbm, a2b_hbm, out_hbm, idxb, rows, outs, semg,
                        sems):
    wid = lax.axis_index("s") * NC + lax.axis_index("c")
    blk0 = wid * GS_K
    pltpu.sync_copy(a2b_hbm.at[pl.ds(blk0, GS_K)], idxb)

    def g_desc(k, j):
        return pltpu.make_async_copy(
            msg_hbm.at[idxb.at[k]], rows[j], semg[j])

    def s_desc(blk, j):
        return pltpu.make_async_copy(
            outs[j], out_hbm.at[pl.ds(blk * A_BLK, A_BLK)], sems[j])

    for k in range(GS_RING - 1):
        g_desc(k, k).start()

    def quad(kq, _):
        for b in range(GS_RING):
            k = kq * GS_RING + b
            blk = blk0 + k

            @pl.when(k + GS_RING - 1 < GS_K)
            def _():
                @pl.when((k >= 1) & (blk - 1 < GS_NBLK))
                def _():
                    s_desc(0, (b + GS_RING - 1) % GS_RING).wait()

                g_desc(k + GS_RING - 1, (b + GS_RING - 1) % GS_RING).start()

            @pl.when(k < GS_K)
            def _():
                g_desc(k, b).wait()

                @pl.when(blk < GS_NBLK)
                def _():
                    def per_atom(a, _):
                        base = a * MAX_NB
                        for c in range(H // 16):
                            acc = rows[b][base, pl.ds(c * 16, 16)]
                            for r in range(1, MAX_NB):
                                acc = acc + rows[b][base + r,
                                                   pl.ds(c * 16, 16)]
                            outs[b][a, pl.ds(c * 16, 16)] = acc
                        return _

                    lax.fori_loop(0, A_BLK, per_atom, None)
                    s_desc(blk, b).start()

        return _

    lax.fori_loop(0, (GS_K + GS_RING - 1) // GS_RING, quad, None)
    for t in range(GS_K - GS_RING, GS_K):
        @pl.when(blk0 + t < GS_NBLK)
        def _():
            s_desc(0, t % GS_RING).wait()


_sc_gather_sum = functools.partial(
    pl.kernel,
    out_type=jax.ShapeDtypeStruct((NA, H), jnp.float32),
    mesh=plsc.VectorSubcoreMesh(core_axis_name="c", subcore_axis_name="s"),
    scratch_types=[
        pltpu.VMEM((GS_K, IDX_BLK), jnp.int32),
        [pltpu.VMEM((IDX_BLK, H), jnp.float32) for _ in range(GS_RING)],
        [pltpu.VMEM((A_BLK, H), jnp.float32) for _ in range(GS_RING)],
        [pltpu.SemaphoreType.DMA for _ in range(GS_RING)],
        [pltpu.SemaphoreType.DMA for _ in range(GS_RING)],
    ],
)(_sc_gather_sum_body)


B_BLK = 128
DF_NBLK = E // B_BLK
DF_KMAX = (DF_NBLK + NW - 1) // NW


DF_K = 80
DF_PAD = NW * DF_K * B_BLK
DF_RING = 3


def _sc_diff_body(asum_hbm, msg_hbm, b2a_hbm, b2revb_hbm, out_hbm,
                  iab, irb, ga, gm, semg, sems):
    wid = lax.axis_index("s") * NC + lax.axis_index("c")
    blk0 = wid * DF_K
    pltpu.sync_copy(b2a_hbm.at[pl.ds(blk0, DF_K)], iab)
    pltpu.sync_copy(b2revb_hbm.at[pl.ds(blk0, DF_K)], irb)

    def ga_desc(k, j):
        return pltpu.make_async_copy(
            asum_hbm.at[iab.at[k]], ga[j], semg[j])

    def gm_desc(k, j):
        return pltpu.make_async_copy(
            msg_hbm.at[irb.at[k]], gm[j], semg[j])

    def s_desc(blk, j):
        return pltpu.make_async_copy(
            ga[j], out_hbm.at[pl.ds(blk * B_BLK, B_BLK)], sems[j])

    for k in range(DF_RING - 1):
        ga_desc(k, k).start()
        gm_desc(k, k).start()

    def trip(kt, _):
        for b in range(DF_RING):
            k = kt * DF_RING + b
            blk = blk0 + k
            nb = (b + DF_RING - 1) % DF_RING

            @pl.when(k + DF_RING - 1 < DF_K)
            def _():
                @pl.when((k >= 1) & (blk - 1 < DF_NBLK))
                def _():
                    s_desc(0, nb).wait()

                ga_desc(k + DF_RING - 1, nb).start()
                gm_desc(k + DF_RING - 1, nb).start()

            @pl.when(k < DF_K)
            def _():
                ga_desc(k, b).wait()
                gm_desc(k, b).wait()

                @pl.when(blk < DF_NBLK)
                def _():
                    def comp(r8, _):
                        for rr in range(8):
                            r = r8 * 8 + rr
                            for c in range(H // 16):
                                ga[b][r, pl.ds(c * 16, 16)] = (
                                    ga[b][r, pl.ds(c * 16, 16)]
                                    - gm[b][r, pl.ds(c * 16, 16)]
                                )
                        return _

                    lax.fori_loop(0, B_BLK // 8, comp, None)
                    s_desc(blk, b).start()

        return _

    lax.fori_loop(0, (DF_K + DF_RING - 1) // DF_RING, trip, None)
    for t in range(DF_K - DF_RING, DF_K):
        @pl.when(blk0 + t < DF_NBLK)
        def _():
            s_desc(0, t % DF_RING).wait()


_sc_diff = functools.partial(
    pl.kernel,
    out_type=jax.ShapeDtypeStruct((E, H), jnp.float32),
    mesh=plsc.VectorSubcoreMesh(core_axis_name="c", subcore_axis_name="s"),
    scratch_types=[
        pltpu.VMEM((DF_K, B_BLK), jnp.int32),
        pltpu.VMEM((DF_K, B_BLK), jnp.int32),
        [pltpu.VMEM((B_BLK, H), jnp.float32) for _ in range(DF_RING)],
        [pltpu.VMEM((B_BLK, H), jnp.float32) for _ in range(DF_RING)],
        [pltpu.SemaphoreType.DMA for _ in range(DF_RING)],
        [pltpu.SemaphoreType.DMA for _ in range(DF_RING)],
    ],
)(_sc_diff_body)


TC_R = 6400


def _tc_input_body(fb_ref, wi_ref, inp_ref, msg_ref):
    x = jnp.dot(fb_ref[...], wi_ref[...], preferred_element_type=jnp.float32)
    inp_ref[...] = x
    msg_ref[...] = jnp.maximum(x, 0.0)


def _tc_input(f_bonds, W_i):
    grid = (E // TC_R,)
    return pl.pallas_call(
        _tc_input_body,
        grid=grid,
        in_specs=[
            pl.BlockSpec((TC_R, BOND_FDIM), lambda i: (i, 0)),
            pl.BlockSpec((BOND_FDIM, H), lambda i: (0, 0)),
        ],
        out_specs=[
            pl.BlockSpec((TC_R, H), lambda i: (i, 0)),
            pl.BlockSpec((TC_R, H), lambda i: (i, 0)),
        ],
        out_shape=[
            jax.ShapeDtypeStruct((E, H), jnp.float32),
            jax.ShapeDtypeStruct((E, H), jnp.float32),
        ],
    )(f_bonds, W_i)


def _tc_update_body(d_ref, inp_ref, wh_ref, out_ref):
    x = jnp.dot(d_ref[...], wh_ref[...], preferred_element_type=jnp.float32)
    out_ref[...] = jnp.maximum(inp_ref[...] + x, 0.0)


def _tc_update(dmat, inp, W_h):
    grid = (E // TC_R,)
    return pl.pallas_call(
        _tc_update_body,
        grid=grid,
        in_specs=[
            pl.BlockSpec((TC_R, H), lambda i: (i, 0)),
            pl.BlockSpec((TC_R, H), lambda i: (i, 0)),
            pl.BlockSpec((H, H), lambda i: (0, 0)),
        ],
        out_specs=pl.BlockSpec((TC_R, H), lambda i: (i, 0)),
        out_shape=jax.ShapeDtypeStruct((E, H), jnp.float32),
    )(dmat, inp, W_h)


FIN_R = 2000
FIN_M = FIN_R // MOL_SIZE


def _tc_final_body(fa_ref, as_ref, woa_ref, woh_ref, bo_ref, out_ref):
    ah = jnp.dot(fa_ref[...], woa_ref[...], preferred_element_type=jnp.float32)
    ah = ah + jnp.dot(as_ref[...], woh_ref[...],
                      preferred_element_type=jnp.float32)
    ah = jnp.maximum(ah + bo_ref[...], 0.0)
    rows = lax.broadcasted_iota(jnp.int32, (FIN_M, FIN_R), 0)
    cols = lax.broadcasted_iota(jnp.int32, (FIN_M, FIN_R), 1)
    pool = jnp.where(cols // MOL_SIZE == rows, 1.0 / MOL_SIZE, 0.0)
    out_ref[...] = jnp.dot(pool, ah, preferred_element_type=jnp.float32)


def _tc_final(f_atoms, a_sum, Wo_a, Wo_h, b_o2):
    grid = (NA // FIN_R,)
    return pl.pallas_call(
        _tc_final_body,
        grid=grid,
        in_specs=[
            pl.BlockSpec((FIN_R, ATOM_FDIM), lambda i: (i, 0)),
            pl.BlockSpec((FIN_R, H), lambda i: (i, 0)),
            pl.BlockSpec((ATOM_FDIM, H), lambda i: (0, 0)),
            pl.BlockSpec((H, H), lambda i: (0, 0)),
            pl.BlockSpec((1, H), lambda i: (0, 0)),
        ],
        out_specs=pl.BlockSpec((FIN_M, H), lambda i: (i, 0)),
        out_shape=jax.ShapeDtypeStruct((N_MOLS, H), jnp.float32),
    )(f_atoms, a_sum, Wo_a, Wo_h, b_o2)


def kernel(f_atoms, f_bonds, a2b, b2a, b2revb, W_i, W_h, W_o, b_o):
    a2b_flat = jnp.pad(a2b.reshape(NA * MAX_NB).astype(jnp.int32),
                       (0, GS_PAD - NA * MAX_NB)).reshape(-1, IDX_BLK)
    b2a = jnp.pad(b2a.astype(jnp.int32), (0, DF_PAD - E)).reshape(-1, B_BLK)
    b2revb = jnp.pad(b2revb.astype(jnp.int32),
                     (0, DF_PAD - E)).reshape(-1, B_BLK)

    inp, msg = _tc_input(f_bonds, W_i)
    for _ in range(DEPTH - 1):
        a_sum = _sc_gather_sum(msg, a2b_flat)
        dmat = _sc_diff(a_sum, msg, b2a, b2revb)
        msg = _tc_update(dmat, inp, W_h)

    a_sum = _sc_gather_sum(msg, a2b_flat)
    Wo_a = W_o[:ATOM_FDIM]
    Wo_h = W_o[ATOM_FDIM:]
    return _tc_final(f_atoms, a_sum, Wo_a, Wo_h, b_o.reshape(1, H))


# speedup vs baseline: 1.0009x; 1.0009x over previous
import functools

import jax
import jax.numpy as jnp
from jax import lax
from jax.experimental import pallas as pl
from jax.experimental.pallas import tpu as pltpu
from jax.experimental.pallas import tpu_sc as plsc

E = 320000
NA = 10000
MAX_NB = 32
BOND_FDIM = 144
ATOM_FDIM = 128
H = 128
DEPTH = 5
N_MOLS = 200
MOL_SIZE = 50

NC = 2
NS = 16
NW = NC * NS

A_BLK = 4
IDX_BLK = A_BLK * MAX_NB
GS_KMAX = 80
GS_NBLK = NW * GS_KMAX
GS_RING = 4


def _sc_gather_sum_body(msg_hbm, a2b_hbm, out_hbm, idxs, rows, outs, semg,
                        sems):
    wid = lax.axis_index("s") * NC + lax.axis_index("c")

    def fire(k, j):
        blk = k * NW + wid
        pltpu.sync_copy(a2b_hbm.at[pl.ds(blk * IDX_BLK, IDX_BLK)], idxs[j])
        pltpu.make_async_copy(msg_hbm.at[idxs[j]], rows[j], semg[j]).start()

    def g_wait(j):
        pltpu.make_async_copy(msg_hbm.at[idxs[j]], rows[j], semg[j]).wait()

    def s_desc(k, j):
        blk = k * NW + wid
        return pltpu.make_async_copy(
            outs[j], out_hbm.at[pl.ds(blk * A_BLK, A_BLK)], sems[j])

    for k in range(GS_RING - 1):
        fire(k, k)

    def quad(kq, _):
        for b in range(GS_RING):
            k = kq * GS_RING + b
            fire(k + GS_RING - 1, (b + GS_RING - 1) % GS_RING)
            g_wait(b)

            @pl.when(k >= GS_RING)
            def _():
                s_desc(0, b).wait()

            def per_atom(a, _):
                base = a * MAX_NB
                for c in range(H // 16):
                    acc = rows[b][base, pl.ds(c * 16, 16)]
                    for r in range(1, MAX_NB):
                        acc = acc + rows[b][base + r, pl.ds(c * 16, 16)]
                    outs[b][a, pl.ds(c * 16, 16)] = acc
                return _

            lax.fori_loop(0, A_BLK, per_atom, None)
            s_desc(k, b).start()

        return _

    lax.fori_loop(0, GS_KMAX // GS_RING - 1, quad, None)
    for k in range(GS_KMAX - GS_RING, GS_KMAX):
        b = k % GS_RING
        if k + GS_RING - 1 < GS_KMAX:
            fire(k + GS_RING - 1, (b + GS_RING - 1) % GS_RING)
        g_wait(b)

        @pl.when(k >= GS_RING)
        def _():
            s_desc(0, b).wait()

        def per_atom(a, _):
            base = a * MAX_NB
            for c in range(H // 16):
                acc = rows[b][base, pl.ds(c * 16, 16)]
                for r in range(1, MAX_NB):
                    acc = acc + rows[b][base + r, pl.ds(c * 16, 16)]
                outs[b][a, pl.ds(c * 16, 16)] = acc
            return _

        lax.fori_loop(0, A_BLK, per_atom, None)
        s_desc(k, b).start()
    for t in range(GS_KMAX - GS_RING, GS_KMAX):
        s_desc(0, t % GS_RING).wait()


_sc_gather_sum = functools.partial(
    pl.kernel,
    out_type=jax.ShapeDtypeStruct((GS_NBLK * A_BLK, H), jnp.float32),
    mesh=plsc.VectorSubcoreMesh(core_axis_name="c", subcore_axis_name="s"),
    scratch_types=[
        [pltpu.VMEM((IDX_BLK,), jnp.int32) for _ in range(GS_RING)],
        [pltpu.VMEM((IDX_BLK, H), jnp.float32) for _ in range(GS_RING)],
        [pltpu.VMEM((A_BLK, H), jnp.float32) for _ in range(GS_RING)],
        [pltpu.SemaphoreType.DMA for _ in range(GS_RING)],
        [pltpu.SemaphoreType.DMA for _ in range(GS_RING)],
    ],
)(_sc_gather_sum_body)


B_BLK = 128
DF_KMAX = 81
DF_NBLK = NW * DF_KMAX
DF_RING = 3


def _sc_diff_body(asum_hbm, msg_hbm, b2a_hbm, b2revb_hbm, out_hbm,
                  ias, irs, ga, gm, semg, sems):
    wid = lax.axis_index("s") * NC + lax.axis_index("c")

    def fire(k, j):
        base = (k * NW + wid) * B_BLK
        pltpu.sync_copy(b2a_hbm.at[pl.ds(base, B_BLK)], ias[j])
        pltpu.sync_copy(b2revb_hbm.at[pl.ds(base, B_BLK)], irs[j])
        pltpu.make_async_copy(asum_hbm.at[ias[j]], ga[j], semg[j]).start()
        pltpu.make_async_copy(msg_hbm.at[irs[j]], gm[j], semg[j]).start()

    def g_wait(j):
        pltpu.make_async_copy(asum_hbm.at[ias[j]], ga[j], semg[j]).wait()
        pltpu.make_async_copy(msg_hbm.at[irs[j]], gm[j], semg[j]).wait()

    def s_desc(k, j):
        base = (k * NW + wid) * B_BLK
        return pltpu.make_async_copy(
            ga[j], out_hbm.at[pl.ds(base, B_BLK)], sems[j])

    def compute(b):
        def comp(r8, _):
            for rr in range(8):
                r = r8 * 8 + rr
                for c in range(H // 16):
                    ga[b][r, pl.ds(c * 16, 16)] = (
                        ga[b][r, pl.ds(c * 16, 16)]
                        - gm[b][r, pl.ds(c * 16, 16)]
                    )
            return _

        lax.fori_loop(0, B_BLK // 8, comp, None)

    for k in range(DF_RING - 1):
        fire(k, k)

    def trip(kt, _):
        for b in range(DF_RING):
            k = kt * DF_RING + b

            @pl.when(k >= 1)
            def _():
                s_desc(0, (b + DF_RING - 1) % DF_RING).wait()

            fire(k + DF_RING - 1, (b + DF_RING - 1) % DF_RING)
            g_wait(b)
            compute(b)
            s_desc(k, b).start()

        return _

    lax.fori_loop(0, DF_KMAX // DF_RING - 1, trip, None)
    for k in range(DF_KMAX - DF_RING, DF_KMAX):
        b = k % DF_RING
        s_desc(0, (b + DF_RING - 1) % DF_RING).wait()
        if k + DF_RING - 1 < DF_KMAX:
            fire(k + DF_RING - 1, (b + DF_RING - 1) % DF_RING)
        g_wait(b)
        compute(b)
        s_desc(k, b).start()
    s_desc(0, (DF_KMAX - 1) % DF_RING).wait()


_sc_diff = functools.partial(
    pl.kernel,
    out_type=jax.ShapeDtypeStruct((DF_NBLK * B_BLK, H), jnp.float32),
    mesh=plsc.VectorSubcoreMesh(core_axis_name="c", subcore_axis_name="s"),
    scratch_types=[
        [pltpu.VMEM((B_BLK,), jnp.int32) for _ in range(DF_RING)],
        [pltpu.VMEM((B_BLK,), jnp.int32) for _ in range(DF_RING)],
        [pltpu.VMEM((B_BLK, H), jnp.float32) for _ in range(DF_RING)],
        [pltpu.VMEM((B_BLK, H), jnp.float32) for _ in range(DF_RING)],
        [pltpu.SemaphoreType.DMA for _ in range(DF_RING)],
        [pltpu.SemaphoreType.DMA for _ in range(DF_RING)],
    ],
)(_sc_diff_body)


TC_R = 6400


def _tc_input_body(fb_ref, wi_ref, inp_ref, msg_ref):
    x = jnp.dot(fb_ref[...], wi_ref[...], preferred_element_type=jnp.float32)
    inp_ref[...] = x
    msg_ref[...] = jnp.maximum(x, 0.0)


def _tc_input(f_bonds, W_i):
    grid = (E // TC_R,)
    return pl.pallas_call(
        _tc_input_body,
        grid=grid,
        in_specs=[
            pl.BlockSpec((TC_R, BOND_FDIM), lambda i: (i, 0)),
            pl.BlockSpec((BOND_FDIM, H), lambda i: (0, 0)),
        ],
        out_specs=[
            pl.BlockSpec((TC_R, H), lambda i: (i, 0)),
            pl.BlockSpec((TC_R, H), lambda i: (i, 0)),
        ],
        out_shape=[
            jax.ShapeDtypeStruct((E, H), jnp.float32),
            jax.ShapeDtypeStruct((E, H), jnp.float32),
        ],
    )(f_bonds, W_i)


def _tc_update_body(d_ref, inp_ref, wh_ref, out_ref):
    x = jnp.dot(d_ref[...], wh_ref[...], preferred_element_type=jnp.float32)
    out_ref[...] = jnp.maximum(inp_ref[...] + x, 0.0)


def _tc_update(dmat, inp, W_h):
    grid = (E // TC_R,)
    return pl.pallas_call(
        _tc_update_body,
        grid=grid,
        in_specs=[
            pl.BlockSpec((TC_R, H), lambda i: (i, 0)),
            pl.BlockSpec((TC_R, H), lambda i: (i, 0)),
            pl.BlockSpec((H, H), lambda i: (0, 0)),
        ],
        out_specs=pl.BlockSpec((TC_R, H), lambda i: (i, 0)),
        out_shape=jax.ShapeDtypeStruct((E, H), jnp.float32),
    )(dmat, inp, W_h)


FIN_R = 2000
FIN_M = FIN_R // MOL_SIZE


def _tc_final_body(fa_ref, as_ref, woa_ref, woh_ref, bo_ref, out_ref):
    ah = jnp.dot(fa_ref[...], woa_ref[...], preferred_element_type=jnp.float32)
    ah = ah + jnp.dot(as_ref[...], woh_ref[...],
                      preferred_element_type=jnp.float32)
    ah = jnp.maximum(ah + bo_ref[...], 0.0)
    rows = lax.broadcasted_iota(jnp.int32, (FIN_M, FIN_R), 0)
    cols = lax.broadcasted_iota(jnp.int32, (FIN_M, FIN_R), 1)
    pool = jnp.where(cols // MOL_SIZE == rows, 1.0 / MOL_SIZE, 0.0)
    out_ref[...] = jnp.dot(pool, ah, preferred_element_type=jnp.float32)


def _tc_final(f_atoms, a_sum, Wo_a, Wo_h, b_o2):
    grid = (NA // FIN_R,)
    return pl.pallas_call(
        _tc_final_body,
        grid=grid,
        in_specs=[
            pl.BlockSpec((FIN_R, ATOM_FDIM), lambda i: (i, 0)),
            pl.BlockSpec((FIN_R, H), lambda i: (i, 0)),
            pl.BlockSpec((ATOM_FDIM, H), lambda i: (0, 0)),
            pl.BlockSpec((H, H), lambda i: (0, 0)),
            pl.BlockSpec((1, H), lambda i: (0, 0)),
        ],
        out_specs=pl.BlockSpec((FIN_M, H), lambda i: (i, 0)),
        out_shape=jax.ShapeDtypeStruct((N_MOLS, H), jnp.float32),
    )(f_atoms, a_sum, Wo_a, Wo_h, b_o2)


def kernel(f_atoms, f_bonds, a2b, b2a, b2revb, W_i, W_h, W_o, b_o):
    a2b_flat = jnp.pad(a2b.reshape(NA * MAX_NB).astype(jnp.int32),
                       (0, GS_NBLK * IDX_BLK - NA * MAX_NB))
    b2a = jnp.pad(b2a.astype(jnp.int32), (0, DF_NBLK * B_BLK - E))
    b2revb = jnp.pad(b2revb.astype(jnp.int32), (0, DF_NBLK * B_BLK - E))

    inp, msg = _tc_input(f_bonds, W_i)
    for _ in range(DEPTH - 1):
        a_sum = _sc_gather_sum(msg, a2b_flat)
        dmat = _sc_diff(a_sum, msg, b2a, b2revb)
        msg = _tc_update(dmat, inp, W_h)

    a_sum = _sc_gather_sum(msg, a2b_flat)
    Wo_a = W_o[:ATOM_FDIM]
    Wo_h = W_o[ATOM_FDIM:]
    return _tc_final(f_atoms, a_sum, Wo_a, Wo_h, b_o.reshape(1, H))


# speedup vs baseline: 2.0649x; 2.0629x over previous
import functools

import jax
import jax.numpy as jnp
from jax import lax
from jax.experimental import pallas as pl
from jax.experimental.pallas import tpu as pltpu
from jax.experimental.pallas import tpu_sc as plsc

E = 320000
NA = 10000
MAX_NB = 32
BOND_FDIM = 144
ATOM_FDIM = 128
H = 128
DEPTH = 5
N_MOLS = 200
MOL_SIZE = 50

NC = 2
NS = 16
NW = NC * NS

A_BLK = 4
IDX_BLK = A_BLK * MAX_NB
GS_NBLK = NA // A_BLK
GS_KMAX = (GS_NBLK + NW - 1) // NW


def _sc_gather_sum_body(msg_hbm, a2b_hbm, out_hbm,
                        idx0, idx1, rows0, rows1, outv0, outv1,
                        semg0, semg1, sems0, sems1):
    wid = lax.axis_index("s") * NC + lax.axis_index("c")
    bufs = ((idx0, rows0, outv0, semg0, sems0),
            (idx1, rows1, outv1, semg1, sems1))

    def fire(blk, idx, rows, semg):
        pltpu.sync_copy(a2b_hbm.at[pl.ds(blk * IDX_BLK, IDX_BLK)], idx)
        pltpu.make_async_copy(msg_hbm.at[idx], rows, semg).start()

    fire(wid, idx0, rows0, semg0)

    def pair(k2, _):
        for b in range(2):
            idx, rows, outv, semg, sems = bufs[b]
            nidx, nrows, _, nsemg, _ = bufs[1 - b]
            k = k2 * 2 + b
            blk = k * NW + wid

            @pl.when(blk + NW < GS_NBLK)
            def _():
                fire(blk + NW, nidx, nrows, nsemg)

            @pl.when(blk < GS_NBLK)
            def _():
                pltpu.make_async_copy(msg_hbm.at[idx], rows, semg).wait()

                @pl.when(k >= 2)
                def _():
                    pltpu.make_async_copy(
                        outv, out_hbm.at[pl.ds(0, A_BLK)], sems).wait()

                for a in range(A_BLK):
                    for c in range(H // 16):
                        acc = rows[a * MAX_NB, pl.ds(c * 16, 16)]
                        for r in range(1, MAX_NB):
                            acc = acc + rows[a * MAX_NB + r, pl.ds(c * 16, 16)]
                        outv[a, pl.ds(c * 16, 16)] = acc
                pltpu.make_async_copy(
                    outv, out_hbm.at[pl.ds(blk * A_BLK, A_BLK)], sems).start()

        return _

    lax.fori_loop(0, (GS_KMAX + 1) // 2, pair, None)
    pltpu.make_async_copy(outv0, out_hbm.at[pl.ds(0, A_BLK)], sems0).wait()
    pltpu.make_async_copy(outv1, out_hbm.at[pl.ds(0, A_BLK)], sems1).wait()


_sc_gather_sum = functools.partial(
    pl.kernel,
    out_type=jax.ShapeDtypeStruct((NA, H), jnp.float32),
    mesh=plsc.VectorSubcoreMesh(core_axis_name="c", subcore_axis_name="s"),
    scratch_types=[
        pltpu.VMEM((IDX_BLK,), jnp.int32),
        pltpu.VMEM((IDX_BLK,), jnp.int32),
        pltpu.VMEM((IDX_BLK, H), jnp.float32),
        pltpu.VMEM((IDX_BLK, H), jnp.float32),
        pltpu.VMEM((A_BLK, H), jnp.float32),
        pltpu.VMEM((A_BLK, H), jnp.float32),
        pltpu.SemaphoreType.DMA,
        pltpu.SemaphoreType.DMA,
        pltpu.SemaphoreType.DMA,
        pltpu.SemaphoreType.DMA,
    ],
)(_sc_gather_sum_body)


B_BLK = 128
DF_NBLK = E // B_BLK
DF_KMAX = (DF_NBLK + NW - 1) // NW


def _sc_diff_body(asum_hbm, msg_hbm, b2a_hbm, b2revb_hbm, out_hbm,
                  ia0, ia1, ir0, ir1, ga0, ga1, gm0, gm1,
                  semg0, semg1, sems0, sems1):
    wid = lax.axis_index("s") * NC + lax.axis_index("c")
    bufs = ((ia0, ir0, ga0, gm0, semg0, sems0),
            (ia1, ir1, ga1, gm1, semg1, sems1))

    def fire(blk, ia, ir, ga, gm, semg):
        base = blk * B_BLK
        pltpu.sync_copy(b2a_hbm.at[pl.ds(base, B_BLK)], ia)
        pltpu.sync_copy(b2revb_hbm.at[pl.ds(base, B_BLK)], ir)
        pltpu.make_async_copy(asum_hbm.at[ia], ga, semg).start()
        pltpu.make_async_copy(msg_hbm.at[ir], gm, semg).start()

    fire(wid, ia0, ir0, ga0, gm0, semg0)

    def pair(k2, _):
        for b in range(2):
            ia, ir, ga, gm, semg, sems = bufs[b]
            nia, nir, nga, ngm, nsemg, nsems = bufs[1 - b]
            k = k2 * 2 + b
            blk = k * NW + wid

            @pl.when(blk + NW < DF_NBLK)
            def _():
                @pl.when(k >= 1)
                def _():
                    pltpu.make_async_copy(
                        nga, out_hbm.at[pl.ds(0, B_BLK)], nsems).wait()

                fire(blk + NW, nia, nir, nga, ngm, nsemg)

            @pl.when(blk < DF_NBLK)
            def _():
                pltpu.make_async_copy(asum_hbm.at[ia], ga, semg).wait()
                pltpu.make_async_copy(msg_hbm.at[ir], gm, semg).wait()

                def comp(r8, _):
                    for rr in range(8):
                        r = r8 * 8 + rr
                        for c in range(H // 16):
                            ga[r, pl.ds(c * 16, 16)] = (
                                ga[r, pl.ds(c * 16, 16)]
                                - gm[r, pl.ds(c * 16, 16)]
                            )
                    return _

                lax.fori_loop(0, B_BLK // 8, comp, None)
                pltpu.make_async_copy(
                    ga, out_hbm.at[pl.ds(blk * B_BLK, B_BLK)], sems).start()

        return _

    lax.fori_loop(0, (DF_KMAX + 1) // 2, pair, None)
    pltpu.make_async_copy(ga0, out_hbm.at[pl.ds(0, B_BLK)], sems0).wait()
    pltpu.make_async_copy(ga1, out_hbm.at[pl.ds(0, B_BLK)], sems1).wait()


_sc_diff = functools.partial(
    pl.kernel,
    out_type=jax.ShapeDtypeStruct((E, H), jnp.float32),
    mesh=plsc.VectorSubcoreMesh(core_axis_name="c", subcore_axis_name="s"),
    scratch_types=[
        pltpu.VMEM((B_BLK,), jnp.int32),
        pltpu.VMEM((B_BLK,), jnp.int32),
        pltpu.VMEM((B_BLK,), jnp.int32),
        pltpu.VMEM((B_BLK,), jnp.int32),
        pltpu.VMEM((B_BLK, H), jnp.float32),
        pltpu.VMEM((B_BLK, H), jnp.float32),
        pltpu.VMEM((B_BLK, H), jnp.float32),
        pltpu.VMEM((B_BLK, H), jnp.float32),
        pltpu.SemaphoreType.DMA,
        pltpu.SemaphoreType.DMA,
        pltpu.SemaphoreType.DMA,
        pltpu.SemaphoreType.DMA,
    ],
)(_sc_diff_body)


TC_R = 6400


def _tc_input_body(fb_ref, wi_ref, inp_ref, msg_ref):
    x = jnp.dot(fb_ref[...], wi_ref[...], preferred_element_type=jnp.float32)
    inp_ref[...] = x
    msg_ref[...] = jnp.maximum(x, 0.0)


def _tc_input(f_bonds, W_i):
    grid = (E // TC_R,)
    return pl.pallas_call(
        _tc_input_body,
        grid=grid,
        in_specs=[
            pl.BlockSpec((TC_R, BOND_FDIM), lambda i: (i, 0)),
            pl.BlockSpec((BOND_FDIM, H), lambda i: (0, 0)),
        ],
        out_specs=[
            pl.BlockSpec((TC_R, H), lambda i: (i, 0)),
            pl.BlockSpec((TC_R, H), lambda i: (i, 0)),
        ],
        out_shape=[
            jax.ShapeDtypeStruct((E, H), jnp.float32),
            jax.ShapeDtypeStruct((E, H), jnp.float32),
        ],
    )(f_bonds, W_i)


def _tc_update_body(d_ref, inp_ref, wh_ref, out_ref):
    x = jnp.dot(d_ref[...], wh_ref[...], preferred_element_type=jnp.float32)
    out_ref[...] = jnp.maximum(inp_ref[...] + x, 0.0)


def _tc_update(dmat, inp, W_h):
    grid = (E // TC_R,)
    return pl.pallas_call(
        _tc_update_body,
        grid=grid,
        in_specs=[
            pl.BlockSpec((TC_R, H), lambda i: (i, 0)),
            pl.BlockSpec((TC_R, H), lambda i: (i, 0)),
            pl.BlockSpec((H, H), lambda i: (0, 0)),
        ],
        out_specs=pl.BlockSpec((TC_R, H), lambda i: (i, 0)),
        out_shape=jax.ShapeDtypeStruct((E, H), jnp.float32),
    )(dmat, inp, W_h)


FIN_R = 2000
FIN_M = FIN_R // MOL_SIZE


def _tc_final_body(fa_ref, as_ref, woa_ref, woh_ref, bo_ref, out_ref):
    ah = jnp.dot(fa_ref[...], woa_ref[...], preferred_element_type=jnp.float32)
    ah = ah + jnp.dot(as_ref[...], woh_ref[...],
                      preferred_element_type=jnp.float32)
    ah = jnp.maximum(ah + bo_ref[...], 0.0)
    rows = lax.broadcasted_iota(jnp.int32, (FIN_M, FIN_R), 0)
    cols = lax.broadcasted_iota(jnp.int32, (FIN_M, FIN_R), 1)
    pool = jnp.where(cols // MOL_SIZE == rows, 1.0 / MOL_SIZE, 0.0)
    out_ref[...] = jnp.dot(pool, ah, preferred_element_type=jnp.float32)


def _tc_final(f_atoms, a_sum, Wo_a, Wo_h, b_o2):
    grid = (NA // FIN_R,)
    return pl.pallas_call(
        _tc_final_body,
        grid=grid,
        in_specs=[
            pl.BlockSpec((FIN_R, ATOM_FDIM), lambda i: (i, 0)),
            pl.BlockSpec((FIN_R, H), lambda i: (i, 0)),
            pl.BlockSpec((ATOM_FDIM, H), lambda i: (0, 0)),
            pl.BlockSpec((H, H), lambda i: (0, 0)),
            pl.BlockSpec((1, H), lambda i: (0, 0)),
        ],
        out_specs=pl.BlockSpec((FIN_M, H), lambda i: (i, 0)),
        out_shape=jax.ShapeDtypeStruct((N_MOLS, H), jnp.float32),
    )(f_atoms, a_sum, Wo_a, Wo_h, b_o2)


def kernel(f_atoms, f_bonds, a2b, b2a, b2revb, W_i, W_h, W_o, b_o):
    a2b_flat = a2b.reshape(NA * MAX_NB).astype(jnp.int32)
    b2a = b2a.astype(jnp.int32)
    b2revb = b2revb.astype(jnp.int32)

    inp, msg = _tc_input(f_bonds, W_i)
    for _ in range(DEPTH - 1):
        a_sum = _sc_gather_sum(msg, a2b_flat)
        dmat = _sc_diff(a_sum, msg, b2a, b2revb)
        msg = _tc_update(dmat, inp, W_h)

    a_sum = _sc_gather_sum(msg, a2b_flat)
    Wo_a = W_o[:ATOM_FDIM]
    Wo_h = W_o[ATOM_FDIM:]
    return _tc_final(f_atoms, a_sum, Wo_a, Wo_h, b_o.reshape(1, H))


# speedup vs baseline: 2.6094x; 1.2637x over previous
import functools

import jax
import jax.numpy as jnp
from jax import lax
from jax.experimental import pallas as pl
from jax.experimental.pallas import tpu as pltpu
from jax.experimental.pallas import tpu_sc as plsc

E = 320000
NA = 10000
MAX_NB = 32
BOND_FDIM = 144
ATOM_FDIM = 128
H = 128
DEPTH = 5
N_MOLS = 200
MOL_SIZE = 50

NC = 2
NS = 16
NW = NC * NS
def _bf16_encode(e, o):
    ue = jax.lax.bitcast_convert_type(e, jnp.int32)
    uo = jax.lax.bitcast_convert_type(o, jnp.int32)
    ue = ue + jnp.int32(0x7FFF) + ((ue >> 16) & jnp.int32(1))
    uo = uo + jnp.int32(0x7FFF) + ((uo >> 16) & jnp.int32(1))
    return (uo & jnp.int32(-65536)) | ((ue >> 16) & jnp.int32(0xFFFF))

A_BLK = 8
IDX_BLK = A_BLK * MAX_NB
GS_NBLK = NA // A_BLK
GS_KMAX = (GS_NBLK + NW - 1) // NW


def _sc_gather_sum_body(msg_hbm, a2b_hbm, out_hbm,
                        ia0, ia1, ib0, ib1, rows0, rows1, outv0, outv1,
                        semg0, semg1, sems0, sems1):
    wid = lax.axis_index("s") * NC + lax.axis_index("c")
    bufs = ((ia0, ib0, rows0, outv0, semg0, sems0),
            (ia1, ib1, rows1, outv1, semg1, sems1))

    def fire(blk, ia, ib, rows, semg):
        pltpu.sync_copy(a2b_hbm.at[pl.ds(blk * IDX_BLK, 128)], ia)
        pltpu.sync_copy(a2b_hbm.at[pl.ds(blk * IDX_BLK + 128, 128)], ib)
        pltpu.make_async_copy(
            msg_hbm.at[ia], rows.at[pl.ds(0, 128)], semg).start()
        pltpu.make_async_copy(
            msg_hbm.at[ib], rows.at[pl.ds(128, 128)], semg).start()

    def g_wait(ia, ib, rows, semg):
        pltpu.make_async_copy(
            msg_hbm.at[ia], rows.at[pl.ds(0, 128)], semg).wait()
        pltpu.make_async_copy(
            msg_hbm.at[ib], rows.at[pl.ds(128, 128)], semg).wait()

    fire(wid, ia0, ib0, rows0, semg0)

    def pair(k2, _):
        for b in range(2):
            ia, ib, rows, outv, semg, sems = bufs[b]
            nia, nib, nrows, _, nsemg, _ = bufs[1 - b]
            k = k2 * 2 + b
            blk = k * NW + wid

            @pl.when(blk + NW < GS_NBLK)
            def _():
                fire(blk + NW, nia, nib, nrows, nsemg)

            @pl.when(blk < GS_NBLK)
            def _():
                g_wait(ia, ib, rows, semg)

                @pl.when(k >= 2)
                def _():
                    pltpu.make_async_copy(
                        outv, out_hbm.at[pl.ds(0, A_BLK)], sems).wait()

                def per_atom(a, _):
                    base = a * MAX_NB
                    for c in range(H // 16):
                        acc = rows[base, pl.ds(c * 16, 16)]
                        for r in range(1, MAX_NB):
                            acc = acc + rows[base + r, pl.ds(c * 16, 16)]
                        outv[a, pl.ds(c * 16, 16)] = acc
                    return _

                lax.fori_loop(0, A_BLK, per_atom, None)
                pltpu.make_async_copy(
                    outv, out_hbm.at[pl.ds(blk * A_BLK, A_BLK)], sems).start()

        return _

    lax.fori_loop(0, (GS_KMAX + 1) // 2, pair, None)
    pltpu.make_async_copy(outv0, out_hbm.at[pl.ds(0, A_BLK)], sems0).wait()
    pltpu.make_async_copy(outv1, out_hbm.at[pl.ds(0, A_BLK)], sems1).wait()


_sc_gather_sum = functools.partial(
    pl.kernel,
    out_type=jax.ShapeDtypeStruct((NA, H), jnp.float32),
    mesh=plsc.VectorSubcoreMesh(core_axis_name="c", subcore_axis_name="s"),
    scratch_types=[
        pltpu.VMEM((128,), jnp.int32),
        pltpu.VMEM((128,), jnp.int32),
        pltpu.VMEM((128,), jnp.int32),
        pltpu.VMEM((128,), jnp.int32),
        pltpu.VMEM((IDX_BLK, H), jnp.float32),
        pltpu.VMEM((IDX_BLK, H), jnp.float32),
        pltpu.VMEM((A_BLK, H), jnp.float32),
        pltpu.VMEM((A_BLK, H), jnp.float32),
        pltpu.SemaphoreType.DMA,
        pltpu.SemaphoreType.DMA,
        pltpu.SemaphoreType.DMA,
        pltpu.SemaphoreType.DMA,
    ],
)(_sc_gather_sum_body)


B_BLK = 128
DF_NBLK = E // B_BLK
DF_KMAX = (DF_NBLK + NW - 1) // NW


def _sc_diff_body(asum_hbm, msg_hbm, b2a_hbm, b2revb_hbm, out_hbm,
                  ia0, ia1, ir0, ir1, ga0, ga1, gm0, gm1, dv0, dv1,
                  semg0, semg1, sems0, sems1):
    wid = lax.axis_index("s") * NC + lax.axis_index("c")
    bufs = ((ia0, ir0, ga0, gm0, dv0, semg0, sems0),
            (ia1, ir1, ga1, gm1, dv1, semg1, sems1))

    def fire(blk, ia, ir, ga, gm, semg):
        base = blk * B_BLK
        pltpu.sync_copy(b2a_hbm.at[pl.ds(base, B_BLK)], ia)
        pltpu.sync_copy(b2revb_hbm.at[pl.ds(base, B_BLK)], ir)
        pltpu.make_async_copy(asum_hbm.at[ia], ga, semg).start()
        pltpu.make_async_copy(msg_hbm.at[ir], gm, semg).start()

    fire(wid, ia0, ir0, ga0, gm0, semg0)

    def pair(k2, _):
        for b in range(2):
            ia, ir, ga, gm, dv, semg, sems = bufs[b]
            nia, nir, nga, ngm, ndv, nsemg, nsems = bufs[1 - b]
            k = k2 * 2 + b
            blk = k * NW + wid

            @pl.when(blk + NW < DF_NBLK)
            def _():
                @pl.when(k >= 1)
                def _():
                    pltpu.make_async_copy(
                        ndv, out_hbm.at[pl.ds(0, B_BLK)], nsems).wait()

                fire(blk + NW, nia, nir, nga, ngm, nsemg)

            @pl.when(blk < DF_NBLK)
            def _():
                pltpu.make_async_copy(asum_hbm.at[ia], ga, semg).wait()
                pltpu.make_async_copy(msg_hbm.at[ir], gm, semg).wait()

                def comp(r8, _):
                    for rr in range(8):
                        r = r8 * 8 + rr
                        for c in range(H // 16):
                            dv[r, pl.ds(c * 16, 16)] = (
                                ga[r, pl.ds(c * 16, 16)]
                                - gm[r, pl.ds(c * 16, 16)]
                            )
                    return _

                lax.fori_loop(0, B_BLK // 8, comp, None)
                pltpu.make_async_copy(
                    dv, out_hbm.at[pl.ds(blk * B_BLK, B_BLK)], sems).start()

        return _

    lax.fori_loop(0, (DF_KMAX + 1) // 2, pair, None)
    pltpu.make_async_copy(dv0, out_hbm.at[pl.ds(0, B_BLK)], sems0).wait()
    pltpu.make_async_copy(dv1, out_hbm.at[pl.ds(0, B_BLK)], sems1).wait()


_sc_diff = functools.partial(
    pl.kernel,
    out_type=jax.ShapeDtypeStruct((E, H), jnp.float32),
    mesh=plsc.VectorSubcoreMesh(core_axis_name="c", subcore_axis_name="s"),
    scratch_types=[
        pltpu.VMEM((B_BLK,), jnp.int32),
        pltpu.VMEM((B_BLK,), jnp.int32),
        pltpu.VMEM((B_BLK,), jnp.int32),
        pltpu.VMEM((B_BLK,), jnp.int32),
        pltpu.VMEM((B_BLK, H), jnp.float32),
        pltpu.VMEM((B_BLK, H), jnp.float32),
        pltpu.VMEM((B_BLK, H), jnp.float32),
        pltpu.VMEM((B_BLK, H), jnp.float32),
        pltpu.VMEM((B_BLK, H), jnp.float32),
        pltpu.VMEM((B_BLK, H), jnp.float32),
        pltpu.SemaphoreType.DMA,
        pltpu.SemaphoreType.DMA,
        pltpu.SemaphoreType.DMA,
        pltpu.SemaphoreType.DMA,
    ],
)(_sc_diff_body)


TC_R = 6400


def _tc_input_body(fb_ref, wi_ref, inp_ref, msg_ref):
    x = jnp.dot(fb_ref[...], wi_ref[...], preferred_element_type=jnp.float32)
    inp_ref[...] = x.astype(jnp.bfloat16)
    msg_ref[...] = jnp.maximum(x, 0.0)


def _tc_input(f_bonds, W_i):
    grid = (E // TC_R,)
    return pl.pallas_call(
        _tc_input_body,
        grid=grid,
        in_specs=[
            pl.BlockSpec((TC_R, BOND_FDIM), lambda i: (i, 0)),
            pl.BlockSpec((BOND_FDIM, H), lambda i: (0, 0)),
        ],
        out_specs=[
            pl.BlockSpec((TC_R, H), lambda i: (i, 0)),
            pl.BlockSpec((TC_R, H), lambda i: (i, 0)),
        ],
        out_shape=[
            jax.ShapeDtypeStruct((E, H), jnp.bfloat16),
            jax.ShapeDtypeStruct((E, H), jnp.float32),
        ],
    )(f_bonds, W_i)


def _tc_update_body(d_ref, inp_ref, wh_ref, out_ref):
    x = jnp.dot(d_ref[...], wh_ref[...], preferred_element_type=jnp.float32)
    out_ref[...] = jnp.maximum(inp_ref[...].astype(jnp.float32) + x, 0.0)


def _tc_update(dmat, inp, W_h):
    grid = (E // TC_R,)
    return pl.pallas_call(
        _tc_update_body,
        grid=grid,
        in_specs=[
            pl.BlockSpec((TC_R, H), lambda i: (i, 0)),
            pl.BlockSpec((TC_R, H), lambda i: (i, 0)),
            pl.BlockSpec((H, H), lambda i: (0, 0)),
        ],
        out_specs=pl.BlockSpec((TC_R, H), lambda i: (i, 0)),
        out_shape=jax.ShapeDtypeStruct((E, H), jnp.float32),
    )(dmat, inp, W_h)


FIN_R = 2000
FIN_M = FIN_R // MOL_SIZE


def _tc_final_body(fa_ref, as_ref, woa_ref, woh_ref, bo_ref, out_ref):
    ah = jnp.dot(fa_ref[...], woa_ref[...], preferred_element_type=jnp.float32)
    ah = ah + jnp.dot(as_ref[...], woh_ref[...],
                      preferred_element_type=jnp.float32)
    ah = jnp.maximum(ah + bo_ref[...], 0.0)
    rows = lax.broadcasted_iota(jnp.int32, (FIN_M, FIN_R), 0)
    cols = lax.broadcasted_iota(jnp.int32, (FIN_M, FIN_R), 1)
    pool = jnp.where(cols // MOL_SIZE == rows, 1.0 / MOL_SIZE, 0.0)
    out_ref[...] = jnp.dot(pool, ah, preferred_element_type=jnp.float32)


def _tc_final(f_atoms, a_sum, Wo_a, Wo_h, b_o2):
    grid = (NA // FIN_R,)
    return pl.pallas_call(
        _tc_final_body,
        grid=grid,
        in_specs=[
            pl.BlockSpec((FIN_R, ATOM_FDIM), lambda i: (i, 0)),
            pl.BlockSpec((FIN_R, H), lambda i: (i, 0)),
            pl.BlockSpec((ATOM_FDIM, H), lambda i: (0, 0)),
            pl.BlockSpec((H, H), lambda i: (0, 0)),
            pl.BlockSpec((1, H), lambda i: (0, 0)),
        ],
        out_specs=pl.BlockSpec((FIN_M, H), lambda i: (i, 0)),
        out_shape=jax.ShapeDtypeStruct((N_MOLS, H), jnp.float32),
    )(f_atoms, a_sum, Wo_a, Wo_h, b_o2)


def kernel(f_atoms, f_bonds, a2b, b2a, b2revb, W_i, W_h, W_o, b_o):
    a2b_flat = a2b.reshape(NA * MAX_NB).astype(jnp.int32)
    b2a = b2a.astype(jnp.int32)
    b2revb = b2revb.astype(jnp.int32)

    inp, msg = _tc_input(f_bonds, W_i)
    for _ in range(DEPTH - 1):
        a_sum = _sc_gather_sum(msg, a2b_flat)
        dmat = _sc_diff(a_sum, msg, b2a, b2revb)
        msg = _tc_update(dmat, inp, W_h)

    a_sum = _sc_gather_sum(msg, a2b_flat)
    Wo_a = W_o[:ATOM_FDIM]
    Wo_h = W_o[ATOM_FDIM:]
    return _tc_final(f_atoms, a_sum, Wo_a, Wo_h, b_o.reshape(1, H))
